# baseline (speedup 1.0000x reference)
import functools

import jax
import jax.numpy as jnp
from jax import lax
from jax.experimental import pallas as pl
from jax.experimental.pallas import tpu as pltpu
from jax.experimental.pallas import tpu_sc as plsc

B = 16384
F = 26
V = 100000
D = 16
DENSE = 13
OUT_W = F * D + DENSE

NC = 2
NS = 16
NW = NC * NS
BPW = B // NW
CHUNK_B = 128
NCHUNK = BPW // CHUNK_B
ROWS = F * CHUNK_B
GPF = CHUNK_B
R128 = V // 8

_mesh = plsc.VectorSubcoreMesh(core_axis_name="c", subcore_axis_name="s")


@functools.partial(
    pl.kernel,
    mesh=_mesh,
    out_type=jax.ShapeDtypeStruct((OUT_W, B), jnp.float32),
    scratch_types=[
        pltpu.VMEM((ROWS,), jnp.int32),
        pltpu.VMEM((ROWS,), jnp.int32),
        pltpu.VMEM((GPF, 128), jnp.float32),
        pltpu.VMEM((GPF, 128), jnp.float32),
        pltpu.VMEM((D, CHUNK_B), jnp.float32),
        pltpu.VMEM((D, CHUNK_B), jnp.float32),
        pltpu.VMEM((DENSE, CHUNK_B), jnp.float32),
        pltpu.SemaphoreType.DMA,
        pltpu.SemaphoreType.DMA,
    ],
    compiler_params=pltpu.CompilerParams(needs_layout_passes=False),
)
def _encode(idx_hbm, tab_hbm, dense_t_hbm, out_hbm, idx_v, gv_v, gb0, gb1,
            tb0, tb1, dn, gsem, wsem):
    wid = lax.axis_index("s") * NC + lax.axis_index("c")
    lane = lax.broadcasted_iota(jnp.int32, (16,), 0)
    gb = (gb0, gb1)
    tb = (tb0, tb1)

    def fire(f, b):
        pltpu.async_copy(
            tab_hbm.at[f].at[gv_v.at[pl.ds(f * GPF, GPF)]], gb[b], gsem
        )

    def gwait(f, b):
        pltpu.make_async_copy(
            tab_hbm.at[f].at[gv_v.at[pl.ds(f * GPF, GPF)]], gb[b], gsem
        ).wait()

    def wwait(f, b, base_b):
        pltpu.make_async_copy(
            tb[b], out_hbm.at[pl.ds(f * D, D), pl.ds(base_b, CHUNK_B)], wsem
        ).wait()

    def extract(f, b, base_b):
        @pl.loop(0, GPF // 16)
        def _grp(g):
            raw = idx_v[pl.ds(f * GPF + g * 16, 16)]
            col0 = (raw & 7) << 4
            rows = g * 16 + lane
            for d in range(D):
                vec = plsc.load_gather(gb[b], [rows, col0 + d])
                tb[b][d, pl.ds(g * 16, 16)] = vec

        pltpu.async_copy(
            tb[b], out_hbm.at[pl.ds(f * D, D), pl.ds(base_b, CHUNK_B)], wsem
        )

    @pl.loop(0, NCHUNK)
    def _chunk(c):
        base_b = wid * BPW + c * CHUNK_B
        base_r = (wid * NCHUNK + c) * ROWS
        pltpu.sync_copy(idx_hbm.at[pl.ds(base_r, ROWS)], idx_v)

        @pl.loop(0, ROWS // 16)
        def _shift(i):
            gv_v[pl.ds(i * 16, 16)] = idx_v[pl.ds(i * 16, 16)] >> 3

        pltpu.sync_copy(dense_t_hbm.at[:, pl.ds(base_b, CHUNK_B)], dn)
        pltpu.async_copy(
            dn, out_hbm.at[pl.ds(F * D, DENSE), pl.ds(base_b, CHUNK_B)], wsem
        )

        fire(0, 0)

        @pl.loop(0, F // 2)
        def _pair(p):
            f0 = 2 * p
            f1 = 2 * p + 1
            fire(f1, 1)
            gwait(f0, 0)

            @pl.when(p > 0)
            def _():
                wwait(2 * p - 2, 0, base_b)

            extract(f0, 0, base_b)

            @pl.when(p + 1 < F // 2)
            def _():
                fire(2 * p + 2, 0)

            gwait(f1, 1)

            @pl.when(p > 0)
            def _():
                wwait(2 * p - 1, 1, base_b)

            extract(f1, 1, base_b)

        wwait(F - 2, 0, base_b)
        wwait(F - 1, 1, base_b)
        pltpu.make_async_copy(
            dn, out_hbm.at[pl.ds(F * D, DENSE), pl.ds(base_b, CHUNK_B)], wsem
        ).wait()


def kernel(sparse_indices, dense_x, tables):
    flat_idx = (
        sparse_indices.reshape(NW, NCHUNK, CHUNK_B, F)
        .transpose(0, 1, 3, 2)
        .reshape(NW * NCHUNK * ROWS)
    )
    tab = tables.reshape(F, R128, 128)
    out_t = _encode(flat_idx, tab, dense_x.astype(jnp.float32).T)
    return out_t.T


# speedup vs baseline: 1.0793x; 1.0793x over previous
import functools

import jax
import jax.numpy as jnp
from jax import lax
from jax.experimental import pallas as pl
from jax.experimental.pallas import tpu as pltpu
from jax.experimental.pallas import tpu_sc as plsc

B = 16384
F = 26
V = 100000
D = 16
DENSE = 13
OUT_W = F * D + DENSE

NC = 2
NS = 16
NW = NC * NS
BPW = B // NW
CHUNK_B = 128
NCHUNK = BPW // CHUNK_B
ROWS = F * CHUNK_B

_mesh = plsc.VectorSubcoreMesh(core_axis_name="c", subcore_axis_name="s")


@functools.partial(
    pl.kernel,
    mesh=_mesh,
    out_type=jax.ShapeDtypeStruct((B, OUT_W), jnp.float32),
    scratch_types=[
        pltpu.VMEM((ROWS,), jnp.int32),
        pltpu.VMEM((ROWS,), jnp.int32),
        pltpu.VMEM((ROWS, D), jnp.float32),
        pltpu.VMEM((ROWS, D), jnp.float32),
        pltpu.VMEM((CHUNK_B, DENSE), jnp.float32),
        pltpu.VMEM((CHUNK_B, DENSE), jnp.float32),
        pltpu.SemaphoreType.DMA,
        pltpu.SemaphoreType.DMA,
    ],
    compiler_params=pltpu.CompilerParams(use_tc_tiling_on_sc=False),
)
def _encode(idx_hbm, table_hbm, dense_hbm, out_hbm, idx0, idx1, rows0, rows1,
            dense0, dense1, gsem, wsem):
    wid = lax.axis_index("s") * NC + lax.axis_index("c")
    idx_v = (idx0, idx1)
    rows_v = (rows0, rows1)
    dense_v = (dense0, dense1)

    def stage(c):
        buf = c % 2
        base_b = wid * BPW + c * CHUNK_B
        pltpu.sync_copy(idx_hbm.at[wid, c], idx_v[buf])
        pltpu.async_copy(table_hbm.at[idx_v[buf]], rows_v[buf], gsem)
        pltpu.sync_copy(dense_hbm.at[pl.ds(base_b, CHUNK_B)], dense_v[buf])

    def emit(c):
        buf = c % 2
        base_b = wid * BPW + c * CHUNK_B
        pltpu.make_async_copy(
            table_hbm.at[idx_v[buf]], rows_v[buf], gsem
        ).wait()

        @pl.loop(0, F)
        def _fire_write(f):
            pltpu.async_copy(
                rows_v[buf].at[pl.ds(f * CHUNK_B, CHUNK_B)],
                out_hbm.at[pl.ds(base_b, CHUNK_B), pl.ds(f * D, D)],
                wsem,
            )

        pltpu.async_copy(
            dense_v[buf],
            out_hbm.at[pl.ds(base_b, CHUNK_B), pl.ds(F * D, DENSE)],
            wsem,
        )

    def drain(c):
        buf = c % 2
        base_b = wid * BPW + c * CHUNK_B

        @pl.loop(0, F)
        def _wait_write(f):
            pltpu.make_async_copy(
                rows_v[buf].at[pl.ds(f * CHUNK_B, CHUNK_B)],
                out_hbm.at[pl.ds(base_b, CHUNK_B), pl.ds(f * D, D)],
                wsem,
            ).wait()

        pltpu.make_async_copy(
            dense_v[buf],
            out_hbm.at[pl.ds(base_b, CHUNK_B), pl.ds(F * D, DENSE)],
            wsem,
        ).wait()

    stage(0)
    for c in range(NCHUNK):
        if c + 1 < NCHUNK:
            if c - 1 >= 0:
                drain(c - 1)
            stage(c + 1)
        emit(c)
    drain(NCHUNK - 2)
    drain(NCHUNK - 1)


def kernel(sparse_indices, dense_x, tables):
    flat_idx = sparse_indices + (jnp.arange(F, dtype=jnp.int32) * V)[None, :]
    flat_idx = (
        flat_idx.reshape(NW, NCHUNK, CHUNK_B, F)
        .transpose(0, 1, 3, 2)
        .reshape(NW, NCHUNK, ROWS)
    )
    table2d = tables.reshape(F * V, D)
    return _encode(flat_idx, table2d, dense_x.astype(jnp.float32))
